# Initial kernel scaffold; baseline (speedup 1.0000x reference)
import jax
import jax.numpy as jnp
from jax.experimental import pallas as pl


def kernel(disease_feats, gene_feats, chemical_feats, species_feats, trans_adj_list, pattern_name, P_disease, P_gene, P_chemical, P_species, W_DD):
    raise NotImplementedError("write your pallas kernel here")


# speedup vs baseline: 3.4575x; 3.4575x over previous
import functools

import jax
import jax.numpy as jnp
from jax import lax
from jax.experimental import pallas as pl
from jax.experimental.pallas import tpu as pltpu
from jax.experimental.pallas import tpu_sc as plsc

NC, NS, LANES = 2, 16, 16
NW = NC * NS
BLK = 128
D_OUT = 64


def _tc_tables(x_ref, p_ref, w_ref, a_ref, b_ref):
    f = jnp.dot(x_ref[...], p_ref[...], preferred_element_type=jnp.float32)
    a_ref[...] = jnp.dot(f, w_ref[:D_OUT, :], preferred_element_type=jnp.float32)
    b_ref[...] = jnp.dot(f, w_ref[D_OUT:, :], preferred_element_type=jnp.float32)


def _sc_edge_body(nblk_total, a_hbm, b_hbm, i0_hbm, i1_hbm, out_hbm,
                  i0_v, i1_v, ra_v, rb_v, sem_a, sem_b):
    wid = lax.axis_index("s") * NC + lax.axis_index("c")
    nblk_floor = nblk_total // NW
    rem = nblk_total - nblk_floor * NW
    nb = nblk_floor + jnp.where(wid < rem, 1, 0)

    def block(j, carry):
        off = (wid + j * NW) * BLK
        pltpu.sync_copy(i0_hbm.at[pl.ds(off, BLK)], i0_v)
        pltpu.sync_copy(i1_hbm.at[pl.ds(off, BLK)], i1_v)
        ca = pltpu.async_copy(a_hbm.at[i0_v], ra_v, sem_a)
        cb = pltpu.async_copy(b_hbm.at[i1_v], rb_v, sem_b)
        ca.wait()
        cb.wait()

        def row(r, c2):
            for c in range(D_OUT // LANES):
                sl = pl.ds(c * LANES, LANES)
                s = ra_v[r, sl] + rb_v[r, sl]
                ra_v[r, sl] = jnp.maximum(s, 0.2 * s)
            return c2

        lax.fori_loop(0, BLK, row, 0)
        pltpu.sync_copy(ra_v, out_hbm.at[pl.ds(off, BLK)])
        return carry

    lax.fori_loop(0, nb, block, 0)


def kernel(disease_feats, gene_feats, chemical_feats, species_feats,
           trans_adj_list, pattern_name, P_disease, P_gene, P_chemical,
           P_species, W_DD):
    n, _ = disease_feats.shape
    e = trans_adj_list.shape[1]
    a, b = pl.pallas_call(
        _tc_tables,
        out_shape=[jax.ShapeDtypeStruct((n, D_OUT), jnp.float32)] * 2,
    )(disease_feats, P_disease, W_DD)

    idx0 = trans_adj_list[0]
    idx1 = trans_adj_list[1]
    nblk = e // BLK

    sc = pl.kernel(
        functools.partial(_sc_edge_body, nblk),
        out_type=jax.ShapeDtypeStruct((e, D_OUT), jnp.float32),
        mesh=plsc.VectorSubcoreMesh(core_axis_name="c", subcore_axis_name="s"),
        compiler_params=pltpu.CompilerParams(use_tc_tiling_on_sc=False),
        scratch_types=[
            pltpu.VMEM((BLK,), jnp.int32),
            pltpu.VMEM((BLK,), jnp.int32),
            pltpu.VMEM((BLK, D_OUT), jnp.float32),
            pltpu.VMEM((BLK, D_OUT), jnp.float32),
            pltpu.SemaphoreType.DMA,
            pltpu.SemaphoreType.DMA,
        ],
    )
    return sc(a, b, idx0, idx1)


# speedup vs baseline: 5.2132x; 1.5078x over previous
import functools

import jax
import jax.numpy as jnp
from jax import lax
from jax.experimental import pallas as pl
from jax.experimental.pallas import tpu as pltpu
from jax.experimental.pallas import tpu_sc as plsc

NC, NS, LANES = 2, 16, 16
NW = NC * NS
G = 128
S = 256
NG = S // G
D_OUT = 64


def _tc_tables(x_ref, p_ref, w_ref, a_ref, b_ref):
    f = jnp.dot(x_ref[...], p_ref[...], preferred_element_type=jnp.float32)
    a_ref[...] = jnp.dot(f, w_ref[:D_OUT, :], preferred_element_type=jnp.float32)
    b_ref[...] = jnp.dot(f, w_ref[D_OUT:, :], preferred_element_type=jnp.float32)


def _sc_edge_body(nstages, a_hbm, b_hbm, i0_hbm, i1_hbm, out_hbm,
                  i0_v0, i0_v1, i1_v0, i1_v1, ra_v0, ra_v1, rb_v0, rb_v1,
                  ro_v0, ro_v1, sem_i0, sem_i1, sem_g0, sem_g1, sem_s0, sem_s1):
    i0_v, i1_v = (i0_v0, i0_v1), (i1_v0, i1_v1)
    ra_v, rb_v = (ra_v0, ra_v1), (rb_v0, rb_v1)
    ro_v = (ro_v0, ro_v1)
    sem_i, sem_g, sem_s = (sem_i0, sem_i1), (sem_g0, sem_g1), (sem_s0, sem_s1)

    wid = lax.axis_index("s") * NC + lax.axis_index("c")
    per = nstages // NW
    rem = nstages - per * NW
    nb = per + jnp.where(wid < rem, 1, 0)

    def issue_idx(j, s):
        blk = (wid + j * NW) * NG
        pltpu.async_copy(i0_hbm.at[pl.ds(blk, NG)], i0_v[s], sem_i[s])
        pltpu.async_copy(i1_hbm.at[pl.ds(blk, NG)], i1_v[s], sem_i[s])

    def wait_idx(s):
        pltpu.make_async_copy(i0_hbm.at[pl.ds(0, NG)], i0_v[s], sem_i[s]).wait()
        pltpu.make_async_copy(i1_hbm.at[pl.ds(0, NG)], i1_v[s], sem_i[s]).wait()

    def issue_gather(s):
        for h in range(NG):
            pltpu.async_copy(a_hbm.at[i0_v[s].at[h]],
                             ra_v[s].at[pl.ds(h * G, G)], sem_g[s])
            pltpu.async_copy(b_hbm.at[i1_v[s].at[h]],
                             rb_v[s].at[pl.ds(h * G, G)], sem_g[s])

    def wait_gather(s):
        pltpu.make_async_copy(a_hbm.at[pl.ds(0, S)], ra_v[s], sem_g[s]).wait()
        pltpu.make_async_copy(b_hbm.at[pl.ds(0, S)], rb_v[s], sem_g[s]).wait()

    def compute(s):
        ra, rb, ro = ra_v[s], rb_v[s], ro_v[s]

        @plsc.parallel_loop(0, S, 1, unroll=4)
        def _(r):
            for c in range(D_OUT // LANES):
                sl = pl.ds(c * LANES, LANES)
                v = ra[r, sl] + rb[r, sl]
                ro[r, sl] = jnp.maximum(v, 0.2 * v)

    def issue_store(j, s):
        off = (wid + j * NW) * S
        pltpu.async_copy(ro_v[s], out_hbm.at[pl.ds(off, S)], sem_s[s])

    def wait_store(s):
        pltpu.make_async_copy(ro_v[s], out_hbm.at[pl.ds(0, S)], sem_s[s]).wait()

    issue_idx(0, 0)
    issue_idx(1, 1)
    wait_idx(0)
    issue_gather(0)

    def outer(jj, carry):
        for b in range(2):
            j = jj * 2 + b
            s, o = b, 1 - b

            @pl.when(j < nb)
            def _():
                @pl.when(j + 1 < nb)
                def _():
                    wait_idx(o)
                    issue_gather(o)

                wait_gather(s)

                @pl.when(j + 2 < nb)
                def _():
                    issue_idx(j + 2, s)

                @pl.when(j >= 2)
                def _():
                    wait_store(s)

                compute(s)
                issue_store(j, s)
        return carry

    lax.fori_loop(0, (nb + 1) // 2, outer, 0)
    wait_store(0)
    wait_store(1)


def kernel(disease_feats, gene_feats, chemical_feats, species_feats,
           trans_adj_list, pattern_name, P_disease, P_gene, P_chemical,
           P_species, W_DD):
    n, _ = disease_feats.shape
    e = trans_adj_list.shape[1]
    a, b = pl.pallas_call(
        _tc_tables,
        out_shape=[jax.ShapeDtypeStruct((n, D_OUT), jnp.float32)] * 2,
    )(disease_feats, P_disease, W_DD)

    idx0 = trans_adj_list[0].reshape(e // G, G)
    idx1 = trans_adj_list[1].reshape(e // G, G)
    nstages = e // S

    sc = pl.kernel(
        functools.partial(_sc_edge_body, nstages),
        out_type=jax.ShapeDtypeStruct((e, D_OUT), jnp.float32),
        mesh=plsc.VectorSubcoreMesh(core_axis_name="c", subcore_axis_name="s"),
        compiler_params=pltpu.CompilerParams(use_tc_tiling_on_sc=False),
        scratch_types=[
            pltpu.VMEM((NG, G), jnp.int32),
            pltpu.VMEM((NG, G), jnp.int32),
            pltpu.VMEM((NG, G), jnp.int32),
            pltpu.VMEM((NG, G), jnp.int32),
            pltpu.VMEM((S, D_OUT), jnp.float32),
            pltpu.VMEM((S, D_OUT), jnp.float32),
            pltpu.VMEM((S, D_OUT), jnp.float32),
            pltpu.VMEM((S, D_OUT), jnp.float32),
            pltpu.VMEM((S, D_OUT), jnp.float32),
            pltpu.VMEM((S, D_OUT), jnp.float32),
            pltpu.SemaphoreType.DMA,
            pltpu.SemaphoreType.DMA,
            pltpu.SemaphoreType.DMA,
            pltpu.SemaphoreType.DMA,
            pltpu.SemaphoreType.DMA,
            pltpu.SemaphoreType.DMA,
        ],
    )
    return sc(a, b, idx0, idx1)
